# baseline (speedup 1.0000x reference)
import jax
import jax.numpy as jnp
from jax.experimental import pallas as pl
from jax.experimental.pallas import tpu as pltpu

TOTAL = 32768
D = 256
N_OUT = TOTAL - 2
RS_OUT = TOTAL - 1
FLAT = N_OUT * D
NDMA = 8
_TILES = FLAT // 128
_PER = -(-_TILES // NDMA)
CHUNKS = []
for _k in range(NDMA):
    _o = _k * _PER * 128
    _s = min(_PER * 128, FLAT - _o)
    if _s > 0:
        CHUNKS.append((_o, _s))


def _copy_body(x_ref, rs_ref, data_ref, rs_out_ref, sems):
    copies = []
    for k, (off, sz) in enumerate(CHUNKS):
        cp = pltpu.make_async_copy(
            x_ref.at[pl.ds(off, sz)], data_ref.at[pl.ds(off, sz)],
            sems.at[k])
        cp.start()
        copies.append(cp)
    rs_out_ref[...] = rs_ref[pl.ds(0, RS_OUT)]
    for cp in copies:
        cp.wait()


def kernel(x_data, x_row_splits):
    data_flat, rs = pl.pallas_call(
        _copy_body,
        in_specs=[
            pl.BlockSpec(memory_space=pltpu.MemorySpace.HBM),
            pl.BlockSpec((TOTAL,), lambda: (0,)),
        ],
        out_specs=[
            pl.BlockSpec(memory_space=pltpu.MemorySpace.HBM),
            pl.BlockSpec((RS_OUT,), lambda: (0,)),
        ],
        out_shape=[
            jax.ShapeDtypeStruct((FLAT,), jnp.float32),
            jax.ShapeDtypeStruct((RS_OUT,), jnp.int32),
        ],
        scratch_shapes=[pltpu.SemaphoreType.DMA((NDMA,))],
    )(x_data.reshape(-1), x_row_splits)
    return (data_flat.reshape(N_OUT, D), rs)


# speedup vs baseline: 32.5973x; 32.5973x over previous
import jax
import jax.numpy as jnp
from jax.experimental import pallas as pl
from jax.experimental.pallas import tpu as pltpu

TOTAL = 32768
D = 256
N_OUT = TOTAL - 2
RS_OUT = TOTAL - 1
BLK = 1024


def _copy_body(x_ref, rs_ref, data_ref, rs_out_ref):
    data_ref[...] = x_ref[...]
    i = pl.program_id(0)

    @pl.when(i == 0)
    def _():
        rs_out_ref[...] = rs_ref[pl.ds(0, RS_OUT)]


def kernel(x_data, x_row_splits):
    grid = (pl.cdiv(N_OUT, BLK),)
    data, rs = pl.pallas_call(
        _copy_body,
        grid=grid,
        in_specs=[
            pl.BlockSpec((BLK, D), lambda i: (i, 0)),
            pl.BlockSpec((TOTAL,), lambda i: (0,)),
        ],
        out_specs=[
            pl.BlockSpec((BLK, D), lambda i: (i, 0)),
            pl.BlockSpec((RS_OUT,), lambda i: (0,)),
        ],
        out_shape=[
            jax.ShapeDtypeStruct((N_OUT, D), jnp.float32),
            jax.ShapeDtypeStruct((RS_OUT,), jnp.int32),
        ],
    )(x_data, x_row_splits)
    return (data, rs)


# speedup vs baseline: 44.1702x; 1.3550x over previous
import jax
import jax.numpy as jnp
from jax.experimental import pallas as pl
from jax.experimental.pallas import tpu as pltpu

TOTAL = 32768
D = 256
N_OUT = TOTAL - 2
RS_OUT = TOTAL - 1
BLK = 2048


def _copy_body(x_ref, rs_ref, data_ref, rs_out_ref):
    data_ref[...] = x_ref[...]
    i = pl.program_id(0)

    @pl.when(i == 0)
    def _():
        rs_out_ref[...] = rs_ref[pl.ds(0, RS_OUT)]


def kernel(x_data, x_row_splits):
    grid = (pl.cdiv(N_OUT, BLK),)
    data, rs = pl.pallas_call(
        _copy_body,
        grid=grid,
        in_specs=[
            pl.BlockSpec((BLK, D), lambda i: (i, 0)),
            pl.BlockSpec((TOTAL,), lambda i: (0,)),
        ],
        out_specs=[
            pl.BlockSpec((BLK, D), lambda i: (i, 0)),
            pl.BlockSpec((RS_OUT,), lambda i: (0,)),
        ],
        out_shape=[
            jax.ShapeDtypeStruct((N_OUT, D), jnp.float32),
            jax.ShapeDtypeStruct((RS_OUT,), jnp.int32),
        ],
    )(x_data, x_row_splits)
    return (data, rs)


# speedup vs baseline: 48.1679x; 1.0905x over previous
import jax
import jax.numpy as jnp
from jax.experimental import pallas as pl
from jax.experimental.pallas import tpu as pltpu

TOTAL = 32768
D = 256
N_OUT = TOTAL - 2
RS_OUT = TOTAL - 1
BLK = 4096


def _copy_body(x_ref, rs_ref, data_ref, rs_out_ref):
    data_ref[...] = x_ref[...]
    i = pl.program_id(0)

    @pl.when(i == 0)
    def _():
        rs_out_ref[...] = rs_ref[pl.ds(0, RS_OUT)]


def kernel(x_data, x_row_splits):
    grid = (pl.cdiv(N_OUT, BLK),)
    data, rs = pl.pallas_call(
        _copy_body,
        grid=grid,
        in_specs=[
            pl.BlockSpec((BLK, D), lambda i: (i, 0)),
            pl.BlockSpec((TOTAL,), lambda i: (0,)),
        ],
        out_specs=[
            pl.BlockSpec((BLK, D), lambda i: (i, 0)),
            pl.BlockSpec((RS_OUT,), lambda i: (0,)),
        ],
        out_shape=[
            jax.ShapeDtypeStruct((N_OUT, D), jnp.float32),
            jax.ShapeDtypeStruct((RS_OUT,), jnp.int32),
        ],
    )(x_data, x_row_splits)
    return (data, rs)


# speedup vs baseline: 51.5122x; 1.0694x over previous
import jax
import jax.numpy as jnp
from jax.experimental import pallas as pl
from jax.experimental.pallas import tpu as pltpu

TOTAL = 32768
D = 256
N_OUT = TOTAL - 2
RS_OUT = TOTAL - 1
BLK = 8192


def _copy_body(x_ref, rs_ref, data_ref, rs_out_ref):
    data_ref[...] = x_ref[...]
    i = pl.program_id(0)

    @pl.when(i == 0)
    def _():
        rs_out_ref[...] = rs_ref[pl.ds(0, RS_OUT)]


def kernel(x_data, x_row_splits):
    grid = (pl.cdiv(N_OUT, BLK),)
    data, rs = pl.pallas_call(
        _copy_body,
        grid=grid,
        in_specs=[
            pl.BlockSpec((BLK, D), lambda i: (i, 0)),
            pl.BlockSpec((TOTAL,), lambda i: (0,)),
        ],
        out_specs=[
            pl.BlockSpec((BLK, D), lambda i: (i, 0)),
            pl.BlockSpec((RS_OUT,), lambda i: (0,)),
        ],
        out_shape=[
            jax.ShapeDtypeStruct((N_OUT, D), jnp.float32),
            jax.ShapeDtypeStruct((RS_OUT,), jnp.int32),
        ],
    )(x_data, x_row_splits)
    return (data, rs)


# speedup vs baseline: 51.5392x; 1.0005x over previous
import jax
import jax.numpy as jnp
from jax.experimental import pallas as pl
from jax.experimental.pallas import tpu as pltpu

TOTAL = 32768
D = 256
N_OUT = TOTAL - 2
RS_OUT = TOTAL - 1
BLK = 10928


def _copy_body(x_ref, rs_ref, data_ref, rs_out_ref):
    data_ref[...] = x_ref[...]
    i = pl.program_id(0)

    @pl.when(i == 0)
    def _():
        rs_out_ref[...] = rs_ref[pl.ds(0, RS_OUT)]


def kernel(x_data, x_row_splits):
    grid = (pl.cdiv(N_OUT, BLK),)
    data, rs = pl.pallas_call(
        _copy_body,
        grid=grid,
        in_specs=[
            pl.BlockSpec((BLK, D), lambda i: (i, 0)),
            pl.BlockSpec((TOTAL,), lambda i: (0,)),
        ],
        out_specs=[
            pl.BlockSpec((BLK, D), lambda i: (i, 0)),
            pl.BlockSpec((RS_OUT,), lambda i: (0,)),
        ],
        out_shape=[
            jax.ShapeDtypeStruct((N_OUT, D), jnp.float32),
            jax.ShapeDtypeStruct((RS_OUT,), jnp.int32),
        ],
    )(x_data, x_row_splits)
    return (data, rs)


# speedup vs baseline: 54.1891x; 1.0514x over previous
import jax
import jax.numpy as jnp
from jax.experimental import pallas as pl
from jax.experimental.pallas import tpu as pltpu

TOTAL = 32768
D = 256
N_OUT = TOTAL - 2
RS_OUT = TOTAL - 1
BLK = 14928


def _copy_body(x_ref, rs_ref, data_ref, rs_out_ref):
    data_ref[...] = x_ref[...]
    i = pl.program_id(0)

    @pl.when(i == 0)
    def _():
        rs_out_ref[...] = rs_ref[pl.ds(0, RS_OUT)]


def kernel(x_data, x_row_splits):
    grid = (pl.cdiv(N_OUT, BLK),)
    data, rs = pl.pallas_call(
        _copy_body,
        grid=grid,
        in_specs=[
            pl.BlockSpec((BLK, D), lambda i: (i, 0)),
            pl.BlockSpec((TOTAL,), lambda i: (0,)),
        ],
        out_specs=[
            pl.BlockSpec((BLK, D), lambda i: (i, 0)),
            pl.BlockSpec((RS_OUT,), lambda i: (0,)),
        ],
        out_shape=[
            jax.ShapeDtypeStruct((N_OUT, D), jnp.float32),
            jax.ShapeDtypeStruct((RS_OUT,), jnp.int32),
        ],
    )(x_data, x_row_splits)
    return (data, rs)
